# baseline (speedup 1.0000x reference)
import functools

import jax
import jax.numpy as jnp
from jax import lax
from jax.experimental import pallas as pl
from jax.experimental.pallas import tpu as pltpu
from jax.experimental.pallas import tpu_sc as plsc

NBUF = 4


def _embed(x, table):
    B, L = x.shape
    V, D = table.shape
    info = plsc.get_sparse_core_info()
    nw = info.num_cores * info.num_subcores
    b_per_w = B // nw
    n_groups = b_per_w // NBUF
    mesh = plsc.VectorSubcoreMesh(core_axis_name="c", subcore_axis_name="s")

    @functools.partial(
        pl.kernel,
        mesh=mesh,
        out_type=jax.ShapeDtypeStruct((B, L, D), jnp.float32),
        scratch_types=[
            pltpu.VMEM((b_per_w, L), jnp.int32),
            pltpu.VMEM((NBUF, L, D), jnp.float32),
            pltpu.SemaphoreType.DMA,
            pltpu.SemaphoreType.DMA((NBUF,)),
        ],
        compiler_params=pltpu.CompilerParams(use_tc_tiling_on_sc=False),
    )
    def emb(idx_hbm, tab_hbm, out_hbm, idx_v, rows_v, gsem, wsem):
        wid = lax.axis_index("s") * info.num_cores + lax.axis_index("c")
        base = wid * b_per_w
        pltpu.sync_copy(idx_hbm.at[pl.ds(base, b_per_w), :], idx_v)

        def gather_desc(i, b):
            return pltpu.make_async_copy(
                tab_hbm.at[idx_v.at[i]],
                rows_v.at[b],
                gsem,
            )

        def write_desc(i, b):
            return pltpu.make_async_copy(
                rows_v.at[b],
                out_hbm.at[base + i],
                wsem.at[b],
            )

        def group(j, _):
            i0 = j * NBUF
            for b in range(NBUF):
                i = i0 + b

                @pl.when(j > 0)
                def _():
                    write_desc(i - NBUF, b).wait()

                gather_desc(i, b).start()
                gather_desc(i, b).wait()
                write_desc(i, b).start()

            return 0

        lax.fori_loop(0, n_groups, group, 0)
        for b in range(NBUF):
            write_desc((n_groups - 1) * NBUF + b, b).wait()

    return emb(x, table)


def kernel(x, table):
    return _embed(x, table)
